# baseline (speedup 1.0000x reference)
import functools

import jax
import jax.numpy as jnp
from jax import lax
from jax.experimental import pallas as pl
from jax.experimental.pallas import tpu as pltpu
from jax.experimental.pallas import tpu_sc as plsc

D = 128
L = 16
NJ = D // L
CH = 128
EPS = 1e-12


def _sc_workers():
    try:
        info = plsc.get_sparse_core_info()
        return int(info.num_cores), int(info.num_subcores)
    except Exception:
        return 2, 16


def kernel(input_ids, segment_ids, token_table, segment_table, ln_gamma, ln_beta):
    NC, NS = _sc_workers()
    NW = NC * NS
    BATCH, SEQ = input_ids.shape
    N = BATCH * SEQ
    per_w = N // NW
    nch = per_w // CH

    ids2 = input_ids.reshape(N // CH, CH).astype(jnp.int32)
    sids = segment_ids.reshape(N).astype(jnp.int32)

    mesh = plsc.VectorSubcoreMesh(core_axis_name="c", subcore_axis_name="s")

    @functools.partial(
        pl.kernel,
        out_type=jax.ShapeDtypeStruct((N, D), jnp.float32),
        mesh=mesh,
        compiler_params=pltpu.CompilerParams(needs_layout_passes=False),
        scratch_types=[
            pltpu.VMEM((nch, CH), jnp.int32),
            pltpu.VMEM((per_w,), jnp.int32),
            pltpu.VMEM((CH, D), jnp.float32),
            pltpu.VMEM((CH, D), jnp.float32),
            pltpu.VMEM((2, D), jnp.float32),
            pltpu.VMEM((D,), jnp.float32),
            pltpu.VMEM((D,), jnp.float32),
            pltpu.SemaphoreType.DMA,
            pltpu.SemaphoreType.DMA,
            pltpu.SemaphoreType.DMA,
            pltpu.SemaphoreType.DMA,
        ],
    )
    def emb_kernel(ids_hbm, sids_hbm, table_hbm, segtab_hbm, gamma_hbm, beta_hbm,
                   out_hbm, ids_v, sids_v, rows_a, rows_b, segt_v, g_v, b_v,
                   gsem_a, gsem_b, osem_a, osem_b):
        wid = lax.axis_index("s") * NC + lax.axis_index("c")
        base = wid * per_w

        pltpu.sync_copy(ids_hbm.at[pl.ds(wid * nch, nch)], ids_v)
        pltpu.sync_copy(sids_hbm.at[pl.ds(base, per_w)], sids_v)
        pltpu.sync_copy(segtab_hbm, segt_v)
        pltpu.sync_copy(gamma_hbm, g_v)
        pltpu.sync_copy(beta_hbm, b_v)

        seg0 = [segt_v[0, pl.ds(j * L, L)] for j in range(NJ)]
        seg1 = [segt_v[1, pl.ds(j * L, L)] for j in range(NJ)]
        gam = [g_v[pl.ds(j * L, L)] for j in range(NJ)]
        bet = [b_v[pl.ds(j * L, L)] for j in range(NJ)]

        bufs = (rows_a, rows_b)
        gsems = (gsem_a, gsem_b)
        osems = (osem_a, osem_b)

        def start_gather(c):
            return pltpu.async_copy(
                table_hbm.at[ids_v.at[c]], bufs[c % 2], gsems[c % 2])

        def start_write(c):
            return pltpu.async_copy(
                bufs[c % 2], out_hbm.at[pl.ds(base + c * CH, CH)], osems[c % 2])

        lane15 = jnp.full((L,), L - 1, jnp.int32)
        iota = lax.iota(jnp.int32, L)
        perms = [iota ^ k for k in (1, 2, 4, 8)]

        def vbcast(v, lanes):
            return lax.gather(
                v, lanes[:, None],
                dimension_numbers=lax.GatherDimensionNumbers(
                    offset_dims=(), collapsed_slice_dims=(0,),
                    start_index_map=(0,)),
                slice_sizes=(1,),
                mode=lax.GatherScatterMode.PROMISE_IN_BOUNDS)

        UNROLL = 2

        def compute_chunk(c):
            rows = bufs[c % 2]

            def one_token(t, p):
                x = [rows[t, pl.ds(j * L, L)] for j in range(NJ)]
                x = [x[j] + jnp.where(p, seg1[j], seg0[j]) for j in range(NJ)]
                s = ((x[0] + x[1]) + (x[2] + x[3])) + ((x[4] + x[5]) + (x[6] + x[7]))
                q = [x[j] * x[j] for j in range(NJ)]
                qs = ((q[0] + q[1]) + (q[2] + q[3])) + ((q[4] + q[5]) + (q[6] + q[7]))
                mean_v = vbcast(plsc.cumsum(s), lane15) * (1.0 / D)
                qmean_v = vbcast(plsc.cumsum(qs), lane15) * (1.0 / D)
                a_v = qmean_v - mean_v * mean_v + EPS
                i = plsc.bitcast(a_v, jnp.int32)
                i = jnp.int32(0x5F3759DF) - (i >> 1)
                y = plsc.bitcast(i, jnp.float32)
                ah = a_v * 0.5
                for _ in range(2):
                    y = y * (1.5 - ah * y * y)
                for j in range(NJ):
                    rows[t, pl.ds(j * L, L)] = (x[j] - mean_v) * (y * gam[j]) + bet[j]

            @plsc.parallel_loop(0, CH // UNROLL, unroll=2)
            def body(tt):
                t0 = tt * UNROLL
                sid16 = sids_v[pl.ds(c * CH + ((t0 >> 4) << 4), L)]
                lane_base = jnp.full((L,), t0 & (L - 1), jnp.int32)
                for u in range(UNROLL):
                    sid = vbcast(sid16, lane_base + u)
                    one_token(t0 + u, sid > 0)

        pending_g = {0: start_gather(0)}
        if nch > 1:
            pending_g[1] = start_gather(1)
        pending_o = {}
        for c in range(nch):
            pending_g.pop(c).wait()
            compute_chunk(c)
            pending_o[c] = start_write(c)
            if c + 2 < nch:
                pending_o.pop(c).wait()
                pending_g[c + 2] = start_gather(c + 2)
        for d in pending_o.values():
            d.wait()

    out = emb_kernel(ids2, sids, token_table, segment_table, ln_gamma, ln_beta)
    return out.reshape(BATCH, SEQ, D)


# speedup vs baseline: 1.0635x; 1.0635x over previous
import functools

import jax
import jax.numpy as jnp
from jax import lax
from jax.experimental import pallas as pl
from jax.experimental.pallas import tpu as pltpu
from jax.experimental.pallas import tpu_sc as plsc

D = 128
L = 16
NJ = D // L
CH = 128
EPS = 1e-12


def _sc_workers():
    try:
        info = plsc.get_sparse_core_info()
        return int(info.num_cores), int(info.num_subcores)
    except Exception:
        return 2, 16


def kernel(input_ids, segment_ids, token_table, segment_table, ln_gamma, ln_beta):
    NC, NS = _sc_workers()
    NW = NC * NS
    BATCH, SEQ = input_ids.shape
    N = BATCH * SEQ
    per_w = N // NW
    nch = per_w // CH

    ids2 = input_ids.reshape(N // CH, CH).astype(jnp.int32)
    sids = segment_ids.reshape(N).astype(jnp.int32)

    mesh = plsc.VectorSubcoreMesh(core_axis_name="c", subcore_axis_name="s")

    @functools.partial(
        pl.kernel,
        out_type=jax.ShapeDtypeStruct((N, D), jnp.float32),
        mesh=mesh,
        compiler_params=pltpu.CompilerParams(needs_layout_passes=False),
        scratch_types=[
            pltpu.VMEM((nch, CH), jnp.int32),
            pltpu.VMEM((per_w,), jnp.int32),
            pltpu.VMEM((CH, D), jnp.float32),
            pltpu.VMEM((CH, D), jnp.float32),
            pltpu.VMEM((CH, D), jnp.float32),
            pltpu.VMEM((CH, D), jnp.float32),
            pltpu.VMEM((2, D), jnp.float32),
            pltpu.VMEM((D,), jnp.float32),
            pltpu.VMEM((D,), jnp.float32),
            pltpu.SemaphoreType.DMA,
            pltpu.SemaphoreType.DMA,
            pltpu.SemaphoreType.DMA,
            pltpu.SemaphoreType.DMA,
            pltpu.SemaphoreType.DMA,
            pltpu.SemaphoreType.DMA,
            pltpu.SemaphoreType.DMA,
            pltpu.SemaphoreType.DMA,
        ],
    )
    def emb_kernel(ids_hbm, sids_hbm, table_hbm, segtab_hbm, gamma_hbm, beta_hbm,
                   out_hbm, ids_v, sids_v, rows_0, rows_1, rows_2, rows_3,
                   segt_v, g_v, b_v,
                   gsem_0, gsem_1, gsem_2, gsem_3,
                   osem_0, osem_1, osem_2, osem_3):
        wid = lax.axis_index("s") * NC + lax.axis_index("c")
        base = wid * per_w

        pltpu.sync_copy(ids_hbm.at[pl.ds(wid * nch, nch)], ids_v)
        pltpu.sync_copy(sids_hbm.at[pl.ds(base, per_w)], sids_v)
        pltpu.sync_copy(segtab_hbm, segt_v)
        pltpu.sync_copy(gamma_hbm, g_v)
        pltpu.sync_copy(beta_hbm, b_v)

        seg0 = [segt_v[0, pl.ds(j * L, L)] for j in range(NJ)]
        seg1 = [segt_v[1, pl.ds(j * L, L)] for j in range(NJ)]
        gam = [g_v[pl.ds(j * L, L)] for j in range(NJ)]
        bet = [b_v[pl.ds(j * L, L)] for j in range(NJ)]

        NB = 4
        bufs = (rows_0, rows_1, rows_2, rows_3)
        gsems = (gsem_0, gsem_1, gsem_2, gsem_3)
        osems = (osem_0, osem_1, osem_2, osem_3)

        def start_gather(c):
            return pltpu.async_copy(
                table_hbm.at[ids_v.at[c]], bufs[c % NB], gsems[c % NB])

        def start_write(c):
            return pltpu.async_copy(
                bufs[c % NB], out_hbm.at[pl.ds(base + c * CH, CH)], osems[c % NB])

        lane15 = jnp.full((L,), L - 1, jnp.int32)
        iota = lax.iota(jnp.int32, L)
        perms = [iota ^ k for k in (1, 2, 4, 8)]

        def vbcast(v, lanes):
            return lax.gather(
                v, lanes[:, None],
                dimension_numbers=lax.GatherDimensionNumbers(
                    offset_dims=(), collapsed_slice_dims=(0,),
                    start_index_map=(0,)),
                slice_sizes=(1,),
                mode=lax.GatherScatterMode.PROMISE_IN_BOUNDS)

        UNROLL = 2

        def compute_chunk(c):
            rows = bufs[c % NB]

            def one_token(t, p):
                x = [rows[t, pl.ds(j * L, L)] for j in range(NJ)]
                x = [x[j] + jnp.where(p, seg1[j], seg0[j]) for j in range(NJ)]
                s = ((x[0] + x[1]) + (x[2] + x[3])) + ((x[4] + x[5]) + (x[6] + x[7]))
                q = [x[j] * x[j] for j in range(NJ)]
                qs = ((q[0] + q[1]) + (q[2] + q[3])) + ((q[4] + q[5]) + (q[6] + q[7]))
                mean_v = vbcast(plsc.cumsum(s), lane15) * (1.0 / D)
                qmean_v = vbcast(plsc.cumsum(qs), lane15) * (1.0 / D)
                a_v = qmean_v - mean_v * mean_v + EPS
                i = plsc.bitcast(a_v, jnp.int32)
                i = jnp.int32(0x5F3759DF) - (i >> 1)
                y = plsc.bitcast(i, jnp.float32)
                ah = a_v * 0.5
                for _ in range(2):
                    y = y * (1.5 - ah * y * y)
                for j in range(NJ):
                    rows[t, pl.ds(j * L, L)] = (x[j] - mean_v) * (y * gam[j]) + bet[j]

            def body(tt, carry):
                t0 = tt * UNROLL
                sid16 = sids_v[pl.ds(c * CH + ((t0 >> 4) << 4), L)]
                lane_base = jnp.full((L,), t0 & (L - 1), jnp.int32)
                for u in range(UNROLL):
                    sid = vbcast(sid16, lane_base + u)
                    one_token(t0 + u, sid > 0)
                return carry

            lax.fori_loop(0, CH // UNROLL, body, 0)

        pending_g = {0: start_gather(0)}
        if nch > 1:
            pending_g[1] = start_gather(1)
        pending_o = {}
        for c in range(nch):
            if c + 2 < nch:
                if c - 2 >= 0:
                    pending_o.pop(c - 2).wait()
                pending_g[c + 2] = start_gather(c + 2)
            pending_g.pop(c).wait()
            compute_chunk(c)
            pending_o[c] = start_write(c)
        for d in pending_o.values():
            d.wait()

    out = emb_kernel(ids2, sids, token_table, segment_table, ln_gamma, ln_beta)
    return out.reshape(BATCH, SEQ, D)


# speedup vs baseline: 1.1441x; 1.0758x over previous
import functools

import jax
import jax.numpy as jnp
from jax import lax
from jax.experimental import pallas as pl
from jax.experimental.pallas import tpu as pltpu
from jax.experimental.pallas import tpu_sc as plsc

D = 128
L = 16
NJ = D // L
CH = 128
EPS = 1e-12
NEWTON = 1


def _sc_workers():
    try:
        info = plsc.get_sparse_core_info()
        return int(info.num_cores), int(info.num_subcores)
    except Exception:
        return 2, 16


def kernel(input_ids, segment_ids, token_table, segment_table, ln_gamma, ln_beta):
    NC, NS = _sc_workers()
    NW = NC * NS
    BATCH, SEQ = input_ids.shape
    N = BATCH * SEQ
    per_w = N // NW
    nch = per_w // CH

    ids2 = input_ids.reshape(N // CH, CH).astype(jnp.int32)
    sids = segment_ids.reshape(N).astype(jnp.int32)

    mesh = plsc.VectorSubcoreMesh(core_axis_name="c", subcore_axis_name="s")

    @functools.partial(
        pl.kernel,
        out_type=jax.ShapeDtypeStruct((N, D), jnp.float32),
        mesh=mesh,
        compiler_params=pltpu.CompilerParams(needs_layout_passes=False),
        scratch_types=[
            pltpu.VMEM((nch, CH), jnp.int32),
            pltpu.VMEM((per_w,), jnp.int32),
            pltpu.VMEM((CH, D), jnp.float32),
            pltpu.VMEM((CH, D), jnp.float32),
            pltpu.VMEM((CH, D), jnp.float32),
            pltpu.VMEM((CH, D), jnp.float32),
            pltpu.VMEM((2, D), jnp.float32),
            pltpu.VMEM((D,), jnp.float32),
            pltpu.VMEM((D,), jnp.float32),
            pltpu.SemaphoreType.DMA,
            pltpu.SemaphoreType.DMA,
            pltpu.SemaphoreType.DMA,
            pltpu.SemaphoreType.DMA,
            pltpu.SemaphoreType.DMA,
            pltpu.SemaphoreType.DMA,
            pltpu.SemaphoreType.DMA,
            pltpu.SemaphoreType.DMA,
        ],
    )
    def emb_kernel(ids_hbm, sids_hbm, table_hbm, segtab_hbm, gamma_hbm, beta_hbm,
                   out_hbm, ids_v, sids_v, rows_0, rows_1, rows_2, rows_3,
                   segt_v, g_v, b_v,
                   gsem_0, gsem_1, gsem_2, gsem_3,
                   osem_0, osem_1, osem_2, osem_3):
        wid = lax.axis_index("s") * NC + lax.axis_index("c")
        base = wid * per_w

        pltpu.sync_copy(ids_hbm.at[pl.ds(wid * nch, nch)], ids_v)
        pltpu.sync_copy(sids_hbm.at[pl.ds(base, per_w)], sids_v)
        pltpu.sync_copy(segtab_hbm, segt_v)
        pltpu.sync_copy(gamma_hbm, g_v)
        pltpu.sync_copy(beta_hbm, b_v)

        seg0 = [segt_v[0, pl.ds(j * L, L)] for j in range(NJ)]
        seg1 = [segt_v[1, pl.ds(j * L, L)] for j in range(NJ)]
        gam = [g_v[pl.ds(j * L, L)] for j in range(NJ)]
        bet = [b_v[pl.ds(j * L, L)] for j in range(NJ)]

        NB = 4
        bufs = (rows_0, rows_1, rows_2, rows_3)
        gsems = (gsem_0, gsem_1, gsem_2, gsem_3)
        osems = (osem_0, osem_1, osem_2, osem_3)

        def start_gather(c):
            return pltpu.async_copy(
                table_hbm.at[ids_v.at[c]], bufs[c % NB], gsems[c % NB])

        def start_write(c):
            return pltpu.async_copy(
                bufs[c % NB], out_hbm.at[pl.ds(base + c * CH, CH)], osems[c % NB])

        lane15 = jnp.full((L,), L - 1, jnp.int32)
        iota = lax.iota(jnp.int32, L)
        perms = [iota ^ k for k in (1, 2, 4, 8)]

        def vbcast(v, lanes):
            return lax.gather(
                v, lanes[:, None],
                dimension_numbers=lax.GatherDimensionNumbers(
                    offset_dims=(), collapsed_slice_dims=(0,),
                    start_index_map=(0,)),
                slice_sizes=(1,),
                mode=lax.GatherScatterMode.PROMISE_IN_BOUNDS)

        UNROLL = 4

        def compute_chunk(c):
            rows = bufs[c % NB]

            def one_token(t, p):
                x = [rows[t, pl.ds(j * L, L)] for j in range(NJ)]
                x = [x[j] + jnp.where(p, seg1[j], seg0[j]) for j in range(NJ)]
                s = ((x[0] + x[1]) + (x[2] + x[3])) + ((x[4] + x[5]) + (x[6] + x[7]))
                q = [x[j] * x[j] for j in range(NJ)]
                qs = ((q[0] + q[1]) + (q[2] + q[3])) + ((q[4] + q[5]) + (q[6] + q[7]))
                mean_v = vbcast(plsc.cumsum(s), lane15) * (1.0 / D)
                qmean_v = vbcast(plsc.cumsum(qs), lane15) * (1.0 / D)
                a_v = qmean_v - mean_v * mean_v + EPS
                i = plsc.bitcast(a_v, jnp.int32)
                i = jnp.int32(0x5F3759DF) - (i >> 1)
                y = plsc.bitcast(i, jnp.float32)
                ah = a_v * 0.5
                for _ in range(NEWTON):
                    y = y * (1.5 - ah * y * y)
                for j in range(NJ):
                    rows[t, pl.ds(j * L, L)] = (x[j] - mean_v) * (y * gam[j]) + bet[j]

            def body(tt, carry):
                t0 = tt * UNROLL
                sid16 = sids_v[pl.ds(c * CH + ((t0 >> 4) << 4), L)]
                lane_base = jnp.full((L,), t0 & (L - 1), jnp.int32)
                for u in range(UNROLL):
                    sid = vbcast(sid16, lane_base + u)
                    one_token(t0 + u, sid > 0)
                return carry

            lax.fori_loop(0, CH // UNROLL, body, 0)

        pending_g = {0: start_gather(0)}
        if nch > 1:
            pending_g[1] = start_gather(1)
        pending_o = {}
        for c in range(nch):
            if c + 2 < nch:
                if c - 2 >= 0:
                    pending_o.pop(c - 2).wait()
                pending_g[c + 2] = start_gather(c + 2)
            pending_g.pop(c).wait()
            compute_chunk(c)
            pending_o[c] = start_write(c)
        for d in pending_o.values():
            d.wait()

    out = emb_kernel(ids2, sids, token_table, segment_table, ln_gamma, ln_beta)
    return out.reshape(BATCH, SEQ, D)


# speedup vs baseline: 1.1855x; 1.0361x over previous
import functools

import jax
import jax.numpy as jnp
from jax import lax
from jax.experimental import pallas as pl
from jax.experimental.pallas import tpu as pltpu
from jax.experimental.pallas import tpu_sc as plsc

D = 128
L = 16
NJ = D // L
CH = 128
EPS = 1e-12
NEWTON = 1


def _sc_workers():
    try:
        info = plsc.get_sparse_core_info()
        return int(info.num_cores), int(info.num_subcores)
    except Exception:
        return 2, 16


def kernel(input_ids, segment_ids, token_table, segment_table, ln_gamma, ln_beta):
    NC, NS = _sc_workers()
    NW = NC * NS
    BATCH, SEQ = input_ids.shape
    N = BATCH * SEQ
    per_w = N // NW
    nch = per_w // CH

    ids2 = input_ids.reshape(N // CH, CH).astype(jnp.int32)
    sids = segment_ids.reshape(N).astype(jnp.int32)

    mesh = plsc.VectorSubcoreMesh(core_axis_name="c", subcore_axis_name="s")

    @functools.partial(
        pl.kernel,
        out_type=jax.ShapeDtypeStruct((N, D), jnp.float32),
        mesh=mesh,
        compiler_params=pltpu.CompilerParams(needs_layout_passes=False),
        scratch_types=[
            pltpu.VMEM((nch, CH), jnp.int32),
            pltpu.VMEM((per_w,), jnp.int32),
            pltpu.VMEM((CH, D), jnp.float32),
            pltpu.VMEM((CH, D), jnp.float32),
            pltpu.VMEM((CH, D), jnp.float32),
            pltpu.VMEM((CH, D), jnp.float32),
            pltpu.VMEM((2, D), jnp.float32),
            pltpu.VMEM((D,), jnp.float32),
            pltpu.VMEM((D,), jnp.float32),
            pltpu.SemaphoreType.DMA,
            pltpu.SemaphoreType.DMA,
            pltpu.SemaphoreType.DMA,
            pltpu.SemaphoreType.DMA,
            pltpu.SemaphoreType.DMA,
            pltpu.SemaphoreType.DMA,
            pltpu.SemaphoreType.DMA,
            pltpu.SemaphoreType.DMA,
        ],
    )
    def emb_kernel(ids_hbm, sids_hbm, table_hbm, segtab_hbm, gamma_hbm, beta_hbm,
                   out_hbm, ids_v, sids_v, rows_0, rows_1, rows_2, rows_3,
                   segt_v, g_v, b_v,
                   gsem_0, gsem_1, gsem_2, gsem_3,
                   osem_0, osem_1, osem_2, osem_3):
        wid = lax.axis_index("s") * NC + lax.axis_index("c")
        base = wid * per_w

        stage = [
            pltpu.async_copy(ids_hbm.at[pl.ds(wid * nch, nch)], ids_v, gsem_0),
            pltpu.async_copy(sids_hbm.at[pl.ds(base, per_w)], sids_v, gsem_1),
            pltpu.async_copy(segtab_hbm, segt_v, gsem_2),
            pltpu.async_copy(gamma_hbm, g_v, gsem_3),
            pltpu.async_copy(beta_hbm, b_v, osem_0),
        ]
        for dsc in stage:
            dsc.wait()

        seg0 = [segt_v[0, pl.ds(j * L, L)] for j in range(NJ)]
        seg1 = [segt_v[1, pl.ds(j * L, L)] for j in range(NJ)]
        gam = [g_v[pl.ds(j * L, L)] for j in range(NJ)]
        bet = [b_v[pl.ds(j * L, L)] for j in range(NJ)]

        NB = 4
        bufs = (rows_0, rows_1, rows_2, rows_3)
        gsems = (gsem_0, gsem_1, gsem_2, gsem_3)
        osems = (osem_0, osem_1, osem_2, osem_3)

        def start_gather(c):
            return pltpu.async_copy(
                table_hbm.at[ids_v.at[c]], bufs[c % NB], gsems[c % NB])

        def start_write(c):
            return pltpu.async_copy(
                bufs[c % NB], out_hbm.at[pl.ds(base + c * CH, CH)], osems[c % NB])

        lane15 = jnp.full((L,), L - 1, jnp.int32)
        iota = lax.iota(jnp.int32, L)
        perms = [iota ^ k for k in (1, 2, 4, 8)]

        def vbcast(v, lanes):
            return lax.gather(
                v, lanes[:, None],
                dimension_numbers=lax.GatherDimensionNumbers(
                    offset_dims=(), collapsed_slice_dims=(0,),
                    start_index_map=(0,)),
                slice_sizes=(1,),
                mode=lax.GatherScatterMode.PROMISE_IN_BOUNDS)

        UNROLL = 4

        def compute_chunk(c):
            rows = bufs[c % NB]

            def one_token(t, p):
                x = [rows[t, pl.ds(j * L, L)] for j in range(NJ)]
                x = [x[j] + jnp.where(p, seg1[j], seg0[j]) for j in range(NJ)]
                s = ((x[0] + x[1]) + (x[2] + x[3])) + ((x[4] + x[5]) + (x[6] + x[7]))
                q = [x[j] * x[j] for j in range(NJ)]
                qs = ((q[0] + q[1]) + (q[2] + q[3])) + ((q[4] + q[5]) + (q[6] + q[7]))
                mean_v = vbcast(plsc.cumsum(s), lane15) * (1.0 / D)
                qmean_v = vbcast(plsc.cumsum(qs), lane15) * (1.0 / D)
                a_v = qmean_v - mean_v * mean_v + EPS
                i = plsc.bitcast(a_v, jnp.int32)
                i = jnp.int32(0x5F3759DF) - (i >> 1)
                y = plsc.bitcast(i, jnp.float32)
                ah = a_v * 0.5
                for _ in range(NEWTON):
                    y = y * (1.5 - ah * y * y)
                for j in range(NJ):
                    rows[t, pl.ds(j * L, L)] = (x[j] - mean_v) * (y * gam[j]) + bet[j]

            def body(tt, carry):
                t0 = tt * UNROLL
                sid16 = sids_v[pl.ds(c * CH + ((t0 >> 4) << 4), L)]
                lane_base = jnp.full((L,), t0 & (L - 1), jnp.int32)
                for u in range(UNROLL):
                    sid = vbcast(sid16, lane_base + u)
                    one_token(t0 + u, sid > 0)
                return carry

            lax.fori_loop(0, CH // UNROLL, body, 0)

        pending_g = {0: start_gather(0)}
        if nch > 1:
            pending_g[1] = start_gather(1)
        pending_o = {}
        for c in range(nch):
            if c + 2 < nch:
                if c - 2 >= 0:
                    pending_o.pop(c - 2).wait()
                pending_g[c + 2] = start_gather(c + 2)
            pending_g.pop(c).wait()
            compute_chunk(c)
            pending_o[c] = start_write(c)
        for d in pending_o.values():
            d.wait()

    out = emb_kernel(ids2, sids, token_table, segment_table, ln_gamma, ln_beta)
    return out.reshape(BATCH, SEQ, D)
